# baseline (speedup 1.0000x reference)
import jax
import jax.numpy as jnp
from jax.experimental import pallas as pl
from jax.experimental.pallas import tpu as pltpu

_N = 200000
_L = 12800
_GRID = -(-_N // _L)


def _zero_fill(res_points_ref, res_coors_ref):
    res_points_ref[...] = jnp.zeros(res_points_ref.shape, jnp.float32)
    res_coors_ref[...] = jnp.zeros(res_coors_ref.shape, jnp.int32)


def kernel(points):
    del points
    pts_t, coors_t = pl.pallas_call(
        _zero_fill,
        grid=(_GRID,),
        out_specs=[
            pl.BlockSpec((6, _L), lambda i: (0, i)),
            pl.BlockSpec((4, _L), lambda i: (0, i)),
        ],
        out_shape=[
            jax.ShapeDtypeStruct((6, _N), jnp.float32),
            jax.ShapeDtypeStruct((4, _N), jnp.int32),
        ],
        compiler_params=pltpu.CompilerParams(
            dimension_semantics=("arbitrary",),
        ),
    )()
    return (pts_t.T, coors_t.T)


# speedup vs baseline: 1.7551x; 1.7551x over previous
import jax
import jax.numpy as jnp
from jax.experimental import pallas as pl
from jax.experimental.pallas import tpu as pltpu

_N = 200000
_BL = 102400


def _zero_fill(res_points_ref, res_coors_ref):
    res_points_ref[...] = jnp.zeros(res_points_ref.shape, jnp.float32)
    res_coors_ref[...] = jnp.zeros(res_coors_ref.shape, jnp.int32)


def kernel(points):
    del points
    pts_t, coors_t = pl.pallas_call(
        _zero_fill,
        grid=(2,),
        out_specs=[
            pl.BlockSpec((6, _BL), lambda i: (0, i)),
            pl.BlockSpec((4, _BL), lambda i: (0, i)),
        ],
        out_shape=[
            jax.ShapeDtypeStruct((6, _N), jnp.float32),
            jax.ShapeDtypeStruct((4, _N), jnp.int32),
        ],
        compiler_params=pltpu.CompilerParams(
            dimension_semantics=("parallel",),
        ),
    )()
    return (pts_t.T, coors_t.T)
